# baseline (speedup 1.0000x reference)
import functools

import jax
import jax.numpy as jnp
from jax import lax
from jax.experimental import pallas as pl
from jax.experimental.pallas import tpu as pltpu
from jax.experimental.pallas import tpu_sc as plsc

_EMBED = 64
_HIST = 50
_R = 2
_NBUF = 2

_GDN = lax.GatherDimensionNumbers(
    offset_dims=(), collapsed_slice_dims=(0,), start_index_map=(0,))


def _vgather(v, idx):
    return lax.gather(v, idx[:, None], dimension_numbers=_GDN,
                      slice_sizes=(1,),
                      mode=lax.GatherScatterMode.PROMISE_IN_BOUNDS)


def _sc_lookup(speakers, table):
    nb = speakers.shape[0]
    info = plsc.get_sparse_core_info()
    ncores, nsub = info.num_cores, info.num_subcores
    nw = ncores * nsub
    rows_per_w = nb // nw
    n_groups = rows_per_w // _R
    n_outer = n_groups // _NBUF
    mesh = plsc.VectorSubcoreMesh(core_axis_name="c", subcore_axis_name="s")

    vslices = [(0, range(16)), (16, range(16)), (32, range(16)),
               (_HIST - 16, range(14, 16))]

    @functools.partial(
        pl.kernel,
        mesh=mesh,
        out_type=jax.ShapeDtypeStruct((nb, _HIST, _EMBED), jnp.float32),
        scratch_types=[
            pltpu.VMEM((3, _EMBED), jnp.float32),
            pltpu.VMEM((_NBUF, _R, _HIST), jnp.int32),
            pltpu.VMEM((_R, _HIST, _EMBED), jnp.float32),
            pltpu.VMEM((_R, _HIST, _EMBED), jnp.float32),
            pltpu.SemaphoreType.DMA,
            pltpu.SemaphoreType.DMA,
            pltpu.SemaphoreType.DMA,
            pltpu.SemaphoreType.DMA,
        ],
    )
    def k(spk_hbm, tab_hbm, out_hbm, tab_v, fidx_v, rows0, rows1,
          si0, si1, so0, so1):
        bufs = (rows0, rows1)
        sem_i = (si0, si1)
        sem_o = (so0, so1)
        wid = lax.axis_index("s") * ncores + lax.axis_index("c")
        w_row0 = wid * rows_per_w

        pltpu.sync_copy(tab_hbm, tab_v)
        t1 = [tab_v[1, pl.ds(s * 16, 16)] for s in range(_EMBED // 16)]
        t2 = [tab_v[2, pl.ds(s * 16, 16)] for s in range(_EMBED // 16)]

        def fire_idx(g, b):
            pltpu.async_copy(spk_hbm.at[pl.ds(w_row0 + g * _R, _R)],
                             fidx_v.at[b], sem_i[b])

        fire_idx(0, 0)
        fire_idx(1, 1)

        def body(it, carry):
            for b in range(_NBUF):
                g = it * _NBUF + b
                buf = bufs[b]
                pltpu.make_async_copy(spk_hbm.at[pl.ds(w_row0, _R)],
                                      fidx_v.at[b], sem_i[b]).wait()

                @pl.when(it >= 1)
                def _drain_out():
                    pltpu.make_async_copy(buf, out_hbm.at[pl.ds(w_row0, _R)],
                                          sem_o[b]).wait()

                for r in range(_R):
                    for off, js in vslices:
                        idx16 = fidx_v[b, r, pl.ds(off, 16)]
                        g1 = (idx16 * (2 - idx16)).astype(jnp.float32)
                        g2 = ((idx16 * (idx16 - 1)) >> 1).astype(jnp.float32)
                        for j in js:
                            h = off + j
                            jv = jnp.full((16,), j, jnp.int32)
                            f1 = _vgather(g1, jv)
                            f2 = _vgather(g2, jv)
                            for s in range(_EMBED // 16):
                                buf[r, h, pl.ds(s * 16, 16)] = (
                                    t1[s] * f1 + t2[s] * f2)

                @pl.when(it < n_outer - 1)
                def _prefetch():
                    fire_idx(it * _NBUF + b + _NBUF, b)

                pltpu.async_copy(buf, out_hbm.at[pl.ds(w_row0 + g * _R, _R)],
                                 sem_o[b])
            return carry

        lax.fori_loop(0, n_outer, body, 0)
        for b in range(_NBUF):
            pltpu.make_async_copy(bufs[b], out_hbm.at[pl.ds(w_row0, _R)],
                                  sem_o[b]).wait()

    return k(speakers, table)


def kernel(speakers, table):
    return _sc_lookup(speakers.astype(jnp.int32), table)
